# Initial kernel scaffold; baseline (speedup 1.0000x reference)
import functools

import jax
import jax.numpy as jnp
from jax.experimental import pallas as pl

_N = 10000
_ROW_BLK = 1250


def _proj_kernel(x_ref, w_ref, asrc_ref, adst_ref, h_ref, al_s_ref, al_d_ref):
    h = jnp.dot(x_ref[...], w_ref[...], preferred_element_type=jnp.float32)
    h_ref[...] = h
    al_s_ref[...] = jnp.dot(h, asrc_ref[...], preferred_element_type=jnp.float32)
    al_d_ref[...] = jnp.dot(h, adst_ref[...], preferred_element_type=jnp.float32)


def _gat_project(x, W, a_src, a_dst):
    n, k = x.shape
    m = W.shape[1]
    grid = (n // _ROW_BLK,)
    h, al_s, al_d = pl.pallas_call(
        _proj_kernel,
        grid=grid,
        in_specs=[
            pl.BlockSpec((_ROW_BLK, k), lambda i: (i, 0)),
            pl.BlockSpec((k, m), lambda i: (0, 0)),
            pl.BlockSpec((m, 1), lambda i: (0, 0)),
            pl.BlockSpec((m, 1), lambda i: (0, 0)),
        ],
        out_specs=[
            pl.BlockSpec((_ROW_BLK, m), lambda i: (i, 0)),
            pl.BlockSpec((_ROW_BLK, 1), lambda i: (i, 0)),
            pl.BlockSpec((_ROW_BLK, 1), lambda i: (i, 0)),
        ],
        out_shape=[
            jax.ShapeDtypeStruct((n, m), jnp.float32),
            jax.ShapeDtypeStruct((n, 1), jnp.float32),
            jax.ShapeDtypeStruct((n, 1), jnp.float32),
        ],
    )(x, W, a_src[:, None], a_dst[:, None])
    return h, al_s[:, 0], al_d[:, 0]


def _elu_mm_kernel(x_ref, w_ref, b_ref, o_ref):
    y = jnp.dot(x_ref[...], w_ref[...], preferred_element_type=jnp.float32)
    y = y + b_ref[...]
    o_ref[...] = jnp.where(y > 0, y, jnp.expm1(y))


def _elu_mm(x, W, b):
    n, k = x.shape
    m = W.shape[1]
    return pl.pallas_call(
        _elu_mm_kernel,
        grid=(n // _ROW_BLK,),
        in_specs=[
            pl.BlockSpec((_ROW_BLK, k), lambda i: (i, 0)),
            pl.BlockSpec((k, m), lambda i: (0, 0)),
            pl.BlockSpec((1, m), lambda i: (0, 0)),
        ],
        out_specs=pl.BlockSpec((_ROW_BLK, m), lambda i: (i, 0)),
        out_shape=jax.ShapeDtypeStruct((n, m), jnp.float32),
    )(x, W, b[None, :])


def _segment_softmax(e, seg, num_segments):
    m = jax.ops.segment_max(e, seg, num_segments=num_segments)
    m = jnp.where(jnp.isfinite(m), m, 0.0)
    ex = jnp.exp(e - m[seg])
    s = jax.ops.segment_sum(ex, seg, num_segments=num_segments)
    return ex / (s[seg] + 1e-16)


def _edge_aggregate(h, al_s, al_d, src, dst):
    e = jax.nn.leaky_relu(al_s[src] + al_d[dst], negative_slope=0.2)
    alpha = _segment_softmax(e, dst, _N)
    return jax.ops.segment_sum(h[src] * alpha[:, None], dst, num_segments=_N)


def kernel(features, edge_index, W_in, a_src_in, a_dst_in, enc_W, enc_b,
           dec_W, dec_b, W_out, a_src_out, a_dst_out):
    src = edge_index[0]
    dst = edge_index[1]
    h, al_s, al_d = _gat_project(features, W_in, a_src_in, a_dst_in)
    h1 = _edge_aggregate(h, al_s, al_d, src, dst)
    hidden_state = _elu_mm(h1, enc_W, enc_b)
    d = _elu_mm(hidden_state, dec_W, dec_b)
    h2, bl_s, bl_d = _gat_project(d, W_out, a_src_out, a_dst_out)
    recon = _edge_aggregate(h2, bl_s, bl_d, src, dst)
    return (hidden_state, recon)


# speedup vs baseline: 1.0116x; 1.0116x over previous
import functools

import jax
import jax.numpy as jnp
from jax.experimental import pallas as pl

_N = 10000
_ROW_BLK = 2000


def _proj_kernel(x_ref, w_ref, a_ref, h_ref, al_ref):
    h = jnp.dot(x_ref[...], w_ref[...], preferred_element_type=jnp.float32)
    h_ref[...] = h
    al_ref[...] = jnp.dot(h, a_ref[...], preferred_element_type=jnp.float32)


def _gat_project(x, W, a_src, a_dst):
    n, k = x.shape
    m = W.shape[1]
    a_mat = jnp.zeros((m, 128), jnp.float32)
    a_mat = a_mat.at[:, 0].set(a_src).at[:, 1].set(a_dst)
    h, al = pl.pallas_call(
        _proj_kernel,
        grid=(n // _ROW_BLK,),
        in_specs=[
            pl.BlockSpec((_ROW_BLK, k), lambda i: (i, 0)),
            pl.BlockSpec((k, m), lambda i: (0, 0)),
            pl.BlockSpec((m, 128), lambda i: (0, 0)),
        ],
        out_specs=[
            pl.BlockSpec((_ROW_BLK, m), lambda i: (i, 0)),
            pl.BlockSpec((_ROW_BLK, 128), lambda i: (i, 0)),
        ],
        out_shape=[
            jax.ShapeDtypeStruct((n, m), jnp.float32),
            jax.ShapeDtypeStruct((n, 128), jnp.float32),
        ],
    )(x, W, a_mat)
    return h, al[:, 0], al[:, 1]


def _elu_mm_kernel(x_ref, w_ref, b_ref, o_ref):
    y = jnp.dot(x_ref[...], w_ref[...], preferred_element_type=jnp.float32)
    y = y + b_ref[0:1, :]
    o_ref[...] = jnp.where(y > 0, y, jnp.exp(jnp.minimum(y, 0.0)) - 1.0)


def _elu_mm(x, W, b):
    n, k = x.shape
    m = W.shape[1]
    return pl.pallas_call(
        _elu_mm_kernel,
        grid=(n // _ROW_BLK,),
        in_specs=[
            pl.BlockSpec((_ROW_BLK, k), lambda i: (i, 0)),
            pl.BlockSpec((k, m), lambda i: (0, 0)),
            pl.BlockSpec((8, m), lambda i: (0, 0)),
        ],
        out_specs=pl.BlockSpec((_ROW_BLK, m), lambda i: (i, 0)),
        out_shape=jax.ShapeDtypeStruct((n, m), jnp.float32),
    )(x, W, jnp.broadcast_to(b[None, :], (8, m)))


def _segment_softmax(e, seg, num_segments):
    m = jax.ops.segment_max(e, seg, num_segments=num_segments)
    m = jnp.where(jnp.isfinite(m), m, 0.0)
    ex = jnp.exp(e - m[seg])
    s = jax.ops.segment_sum(ex, seg, num_segments=num_segments)
    return ex / (s[seg] + 1e-16)


def _edge_aggregate(h, al_s, al_d, src, dst):
    e = jax.nn.leaky_relu(al_s[src] + al_d[dst], negative_slope=0.2)
    alpha = _segment_softmax(e, dst, _N)
    return jax.ops.segment_sum(h[src] * alpha[:, None], dst, num_segments=_N)


def kernel(features, edge_index, W_in, a_src_in, a_dst_in, enc_W, enc_b,
           dec_W, dec_b, W_out, a_src_out, a_dst_out):
    src = edge_index[0]
    dst = edge_index[1]
    h, al_s, al_d = _gat_project(features, W_in, a_src_in, a_dst_in)
    h1 = _edge_aggregate(h, al_s, al_d, src, dst)
    hidden_state = _elu_mm(h1, enc_W, enc_b)
    d = _elu_mm(hidden_state, dec_W, dec_b)
    h2, bl_s, bl_d = _gat_project(d, W_out, a_src_out, a_dst_out)
    recon = _edge_aggregate(h2, bl_s, bl_d, src, dst)
    return (hidden_state, recon)


# speedup vs baseline: 8.3680x; 8.2718x over previous
import dataclasses
import functools

import jax
import jax.numpy as jnp
from jax import lax
from jax.experimental import pallas as pl
from jax.experimental.pallas import tpu as pltpu
from jax.experimental.pallas import tpu_sc as plsc

_N = 10000
_NBIN = 10240
_E = 160000
_EPAD = 160256
_PER_TILE = 5008
_NVREG = 313
_BIN_SLICE = _NBIN // 16
_ROW_BLK = 2000

_MESH = plsc.VectorSubcoreMesh(core_axis_name="c", subcore_axis_name="s")

_SC_PARAMS = pltpu.CompilerParams()
for _f, _v in (("needs_layout_passes", False), ("use_tc_tiling_on_sc", False)):
    if _f in pltpu.CompilerParams.__dataclass_fields__:
        _SC_PARAMS = dataclasses.replace(_SC_PARAMS, **{_f: _v})


def _proj_kernel(x_ref, w_ref, a_ref, h_ref, al_ref):
    h = jnp.dot(x_ref[...], w_ref[...], preferred_element_type=jnp.float32)
    h_ref[...] = h
    al_ref[...] = jnp.dot(h, a_ref[...], preferred_element_type=jnp.float32)


def _gat_project(x, W, a_src, a_dst):
    n, k = x.shape
    m = W.shape[1]
    a_mat = jnp.zeros((m, 128), jnp.float32)
    a_mat = a_mat.at[:, 0].set(a_src).at[:, 1].set(a_dst)
    h, al = pl.pallas_call(
        _proj_kernel,
        grid=(n // _ROW_BLK,),
        in_specs=[
            pl.BlockSpec((_ROW_BLK, k), lambda i: (i, 0)),
            pl.BlockSpec((k, m), lambda i: (0, 0)),
            pl.BlockSpec((m, 128), lambda i: (0, 0)),
        ],
        out_specs=[
            pl.BlockSpec((_ROW_BLK, m), lambda i: (i, 0)),
            pl.BlockSpec((_ROW_BLK, 128), lambda i: (i, 0)),
        ],
        out_shape=[
            jax.ShapeDtypeStruct((n, m), jnp.float32),
            jax.ShapeDtypeStruct((n, 128), jnp.float32),
        ],
    )(x, W, a_mat)
    return h, al[:, 0], al[:, 1]


def _elu_mm_kernel(x_ref, w_ref, b_ref, o_ref):
    y = jnp.dot(x_ref[...], w_ref[...], preferred_element_type=jnp.float32)
    y = y + b_ref[0:1, :]
    o_ref[...] = jnp.where(y > 0, y, jnp.exp(jnp.minimum(y, 0.0)) - 1.0)


def _elu_mm(x, W, b):
    n, k = x.shape
    m = W.shape[1]
    return pl.pallas_call(
        _elu_mm_kernel,
        grid=(n // _ROW_BLK,),
        in_specs=[
            pl.BlockSpec((_ROW_BLK, k), lambda i: (i, 0)),
            pl.BlockSpec((k, m), lambda i: (0, 0)),
            pl.BlockSpec((8, m), lambda i: (0, 0)),
        ],
        out_specs=pl.BlockSpec((_ROW_BLK, m), lambda i: (i, 0)),
        out_shape=jax.ShapeDtypeStruct((n, m), jnp.float32),
    )(x, W, jnp.broadcast_to(b[None, :], (8, m)))


def _edge_sc_body(src_hbm, dst_hbm, als_hbm, ald_hbm, ex_hbm, sp_hbm,
                  srcv, dstv, alsv, aldv, exv, binsv, redv, outv, staged):
    c = lax.axis_index("c")
    s = lax.axis_index("s")
    wid = s * 2 + c
    base = wid * _PER_TILE
    pltpu.sync_copy(src_hbm.at[pl.ds(base, _PER_TILE)], srcv)
    pltpu.sync_copy(dst_hbm.at[pl.ds(base, _PER_TILE)], dstv)
    pltpu.sync_copy(als_hbm, alsv)
    pltpu.sync_copy(ald_hbm, aldv)
    zeros16 = jnp.zeros((16,), jnp.float32)

    @pl.loop(0, _NBIN, step=16)
    def _(i):
        binsv[pl.ds(i, 16)] = zeros16

    iota = lax.iota(jnp.int32, 16)
    shift_key = (iota + 15) % 16

    @pl.loop(0, _NVREG)
    def _(j):
        off = j * 16
        sidx = srcv[pl.ds(off, 16)]
        didx = dstv[pl.ds(off, 16)]
        x = plsc.load_gather(alsv, [sidx]) + plsc.load_gather(aldv, [didx])
        ex = jnp.exp(jnp.maximum(x, 0.2 * x))
        gid = (base + off) + iota
        ex = jnp.where(gid < _E, ex, 0.0)
        exv[pl.ds(off, 16)] = ex
        ks, vs = plsc.sort_key_val(didx, ex)
        cs = plsc.cumsum(vs)
        _, ks_n = plsc.sort_key_val(shift_key, ks)
        last = (ks != ks_n) | (iota == 15)
        plsc.addupdate_scatter(binsv, [ks], cs, mask=last)
        plsc.addupdate_scatter(binsv, [ks_n], -cs, mask=last & (iota < 15))

    pltpu.sync_copy(exv, ex_hbm.at[pl.ds(base, _PER_TILE)])
    pltpu.sync_copy(binsv, staged.at[s])
    plsc.subcore_barrier()
    pltpu.sync_copy(staged.at[:, pl.ds(s * _BIN_SLICE, _BIN_SLICE)], redv)

    @pl.loop(0, _BIN_SLICE, step=16)
    def _(k):
        acc = redv[0, pl.ds(k, 16)]
        for r in range(1, 16):
            acc = acc + redv[r, pl.ds(k, 16)]
        outv[pl.ds(k, 16)] = acc

    pltpu.sync_copy(outv, sp_hbm.at[c, pl.ds(s * _BIN_SLICE, _BIN_SLICE)])


def _edge_sc(src_pad, dst_pad, al_s, al_d):
    f32 = jnp.float32
    k = pl.kernel(
        _edge_sc_body,
        out_type=[
            jax.ShapeDtypeStruct((_EPAD,), f32),
            jax.ShapeDtypeStruct((2, _NBIN), f32),
        ],
        mesh=_MESH,
        compiler_params=_SC_PARAMS,
        scratch_types=[
            pltpu.VMEM((_PER_TILE,), jnp.int32),
            pltpu.VMEM((_PER_TILE,), jnp.int32),
            pltpu.VMEM((_N,), f32),
            pltpu.VMEM((_N,), f32),
            pltpu.VMEM((_PER_TILE,), f32),
            pltpu.VMEM((_NBIN,), f32),
            pltpu.VMEM((16, _BIN_SLICE), f32),
            pltpu.VMEM((_BIN_SLICE,), f32),
            pltpu.VMEM_SHARED((16, _NBIN), f32),
        ],
    )
    return k(src_pad, dst_pad, al_s, al_d)


_NBLK = _NVREG
_NROWP = 10240
_ROWS_PER_TILE = _NROWP // 16
_CHUNK = 64


def _agg_sc_body(hc_hbm, srcm_hbm, dstm_hbm, exm_hbm, zr_hbm, part_hbm,
                 srcv, dstv, exv, gb, sb, acc, sem0, sem1, sem2, sem3):
    c = lax.axis_index("c")
    s = lax.axis_index("s")
    wid = s * 2 + c
    pltpu.sync_copy(srcm_hbm.at[wid], srcv)
    pltpu.sync_copy(dstm_hbm.at[wid], dstv)
    pltpu.sync_copy(exm_hbm.at[wid], exv)
    row0 = s * _ROWS_PER_TILE
    pltpu.sync_copy(zr_hbm.at[pl.ds(row0, _ROWS_PER_TILE)],
                    acc.at[pl.ds(row0, _ROWS_PER_TILE)])
    plsc.subcore_barrier()

    sems = [sem0, sem1, sem2, sem3]
    iota = lax.iota(jnp.int32, 16)

    def gather(b, k):
        pltpu.async_copy(hc_hbm.at[srcv.at[b]], gb.at[k], sems[k])

    def process(b, k):
        pltpu.make_async_copy(hc_hbm.at[srcv.at[b]], gb.at[k], sems[k]).wait()
        bvec = iota * 0 + b
        for r in range(16):
            ev = plsc.load_gather(exv, [bvec, jnp.full((16,), r, jnp.int32)])
            for q in range(_CHUNK // 16):
                sl = pl.ds(q * 16, 16)
                sb[r, sl] = gb[k, r, sl] * ev
        pltpu.sync_copy(sb, acc.at[dstv.at[b]], add=True)

    for k in range(4):
        gather(k, k)

    @pl.loop(0, _NBLK - 1, step=4)
    def _(j):
        for k in range(4):
            b = j + k
            process(b, k)

            @pl.when(b + 4 <= _NBLK - 1)
            def _():
                gather(b + 4, k)

    process(_NBLK - 1, (_NBLK - 1) % 4)
    plsc.subcore_barrier()
    pltpu.sync_copy(acc.at[pl.ds(row0, _ROWS_PER_TILE)],
                    part_hbm.at[c, pl.ds(row0, _ROWS_PER_TILE)])


def _agg_sc(h_chunk, srcm, dstm, exm, zeros_chunk):
    f32 = jnp.float32
    k = pl.kernel(
        _agg_sc_body,
        out_type=jax.ShapeDtypeStruct((2, _NROWP, _CHUNK), f32),
        mesh=_MESH,
        compiler_params=_SC_PARAMS,
        scratch_types=[
            pltpu.VMEM((_NVREG, 16), jnp.int32),
            pltpu.VMEM((_NVREG, 16), jnp.int32),
            pltpu.VMEM((_NVREG, 16), f32),
            pltpu.VMEM((4, 16, _CHUNK), f32),
            pltpu.VMEM((16, _CHUNK), f32),
            pltpu.VMEM_SHARED((_NROWP, _CHUNK), f32),
            pltpu.SemaphoreType.DMA,
            pltpu.SemaphoreType.DMA,
            pltpu.SemaphoreType.DMA,
            pltpu.SemaphoreType.DMA,
        ],
    )
    return k(h_chunk, srcm, dstm, exm, zeros_chunk)


def _combine_enc_kernel(p0_ref, p1_ref, p2_ref, p3_ref, p4_ref, p5_ref,
                        p6_ref, p7_ref, r_ref, w_ref, b_ref, o_ref):
    parts = []
    for p in (p0_ref, p1_ref, p2_ref, p3_ref, p4_ref, p5_ref, p6_ref, p7_ref):
        parts.append((p[0] + p[1]) * r_ref[...])
    x = jnp.concatenate(parts, axis=1)
    y = jnp.dot(x, w_ref[...], preferred_element_type=jnp.float32)
    y = y + b_ref[0:1, :]
    o_ref[...] = jnp.where(y > 0, y, jnp.exp(jnp.minimum(y, 0.0)) - 1.0)


def _combine_enc(parts, recip_mat, enc_W, enc_b):
    m = enc_W.shape[1]
    return pl.pallas_call(
        _combine_enc_kernel,
        grid=(_N // _ROW_BLK,),
        in_specs=[pl.BlockSpec((2, _ROW_BLK, _CHUNK), lambda i: (0, i, 0))] * 8
        + [
            pl.BlockSpec((_ROW_BLK, _CHUNK), lambda i: (i, 0)),
            pl.BlockSpec((512, m), lambda i: (0, 0)),
            pl.BlockSpec((8, m), lambda i: (0, 0)),
        ],
        out_specs=pl.BlockSpec((_ROW_BLK, m), lambda i: (i, 0)),
        out_shape=jax.ShapeDtypeStruct((_N, m), jnp.float32),
    )(*parts, recip_mat, enc_W, jnp.broadcast_to(enc_b[None, :], (8, m)))


def _combine_out_kernel(q0_ref, q1_ref, q2_ref, q3_ref, r_ref, o_ref):
    qs = [(q[0] + q[1]) * r_ref[...]
          for q in (q0_ref, q1_ref, q2_ref, q3_ref)]
    o_ref[...] = jnp.concatenate(qs, axis=1)


def _combine_out(parts, recip_mat):
    return pl.pallas_call(
        _combine_out_kernel,
        grid=(_N // _ROW_BLK,),
        in_specs=[pl.BlockSpec((2, _ROW_BLK, _CHUNK), lambda i: (0, i, 0))] * 4
        + [pl.BlockSpec((_ROW_BLK, _CHUNK), lambda i: (i, 0))],
        out_specs=pl.BlockSpec((_ROW_BLK, 256), lambda i: (i, 0)),
        out_shape=jax.ShapeDtypeStruct((_N, 256), jnp.float32),
    )(*parts, recip_mat)


def kernel(features, edge_index, W_in, a_src_in, a_dst_in, enc_W, enc_b,
           dec_W, dec_b, W_out, a_src_out, a_dst_out):
    src = edge_index[0]
    dst = edge_index[1]
    src_pad = jnp.pad(src, (0, _EPAD - _E))
    dst_pad = jnp.pad(dst, (0, _EPAD - _E))
    srcm = src_pad.reshape(32, _NVREG, 16)
    dstm = dst_pad.reshape(32, _NVREG, 16)
    zeros_chunk = jnp.zeros((_NROWP, _CHUNK), jnp.float32)

    h, al_s, al_d = _gat_project(features, W_in, a_src_in, a_dst_in)
    ex1, sp1 = _edge_sc(src_pad, dst_pad, al_s, al_d)
    exm1 = ex1.reshape(32, _NVREG, 16)
    recip1 = 1.0 / (sp1[0, :_N] + sp1[1, :_N] + 1e-16)
    recip1_mat = jnp.broadcast_to(recip1[:, None], (_N, _CHUNK))
    parts1 = [
        _agg_sc(h[:, i * _CHUNK:(i + 1) * _CHUNK], srcm, dstm, exm1,
                zeros_chunk) for i in range(512 // _CHUNK)
    ]
    hidden_state = _combine_enc(parts1, recip1_mat, enc_W, enc_b)

    d = _elu_mm(hidden_state, dec_W, dec_b)
    h2, bl_s, bl_d = _gat_project(d, W_out, a_src_out, a_dst_out)
    ex2, sp2 = _edge_sc(src_pad, dst_pad, bl_s, bl_d)
    exm2 = ex2.reshape(32, _NVREG, 16)
    recip2 = 1.0 / (sp2[0, :_N] + sp2[1, :_N] + 1e-16)
    recip2_mat = jnp.broadcast_to(recip2[:, None], (_N, _CHUNK))
    parts2 = [
        _agg_sc(h2[:, i * _CHUNK:(i + 1) * _CHUNK], srcm, dstm, exm2,
                zeros_chunk) for i in range(256 // _CHUNK)
    ]
    recon = _combine_out(parts2, recip2_mat)
    return (hidden_state, recon)


# speedup vs baseline: 17.0611x; 2.0389x over previous
import dataclasses
import functools

import jax
import jax.numpy as jnp
from jax import lax
from jax.experimental import pallas as pl
from jax.experimental.pallas import tpu as pltpu
from jax.experimental.pallas import tpu_sc as plsc

_N = 10000
_NBIN = 10240
_E = 160000
_EPAD = 160256
_PER_TILE = 5008
_NVREG = 313
_BIN_SLICE = _NBIN // 16
_ROW_BLK = 2000

_MESH = plsc.VectorSubcoreMesh(core_axis_name="c", subcore_axis_name="s")

_SC_PARAMS = pltpu.CompilerParams()
for _f, _v in (("needs_layout_passes", False), ("use_tc_tiling_on_sc", False)):
    if _f in pltpu.CompilerParams.__dataclass_fields__:
        _SC_PARAMS = dataclasses.replace(_SC_PARAMS, **{_f: _v})


def _proj_kernel(x_ref, w_ref, a_ref, h_ref, al_ref):
    h = jnp.dot(x_ref[...], w_ref[...], preferred_element_type=jnp.float32)
    h_ref[...] = h
    al_ref[...] = jnp.dot(h, a_ref[...], preferred_element_type=jnp.float32)


def _gat_project(x, W, a_src, a_dst):
    n, k = x.shape
    m = W.shape[1]
    a_mat = jnp.zeros((m, 128), jnp.float32)
    a_mat = a_mat.at[:, 0].set(a_src).at[:, 1].set(a_dst)
    h, al = pl.pallas_call(
        _proj_kernel,
        grid=(n // _ROW_BLK,),
        in_specs=[
            pl.BlockSpec((_ROW_BLK, k), lambda i: (i, 0)),
            pl.BlockSpec((k, m), lambda i: (0, 0)),
            pl.BlockSpec((m, 128), lambda i: (0, 0)),
        ],
        out_specs=[
            pl.BlockSpec((_ROW_BLK, m), lambda i: (i, 0)),
            pl.BlockSpec((_ROW_BLK, 128), lambda i: (i, 0)),
        ],
        out_shape=[
            jax.ShapeDtypeStruct((n, m), jnp.float32),
            jax.ShapeDtypeStruct((n, 128), jnp.float32),
        ],
    )(x, W, a_mat)
    return h, al[:, 0], al[:, 1]


def _elu_mm_kernel(x_ref, w_ref, b_ref, o_ref):
    y = jnp.dot(x_ref[...], w_ref[...], preferred_element_type=jnp.float32)
    y = y + b_ref[0:1, :]
    o_ref[...] = jnp.where(y > 0, y, jnp.exp(jnp.minimum(y, 0.0)) - 1.0)


def _elu_mm(x, W, b):
    n, k = x.shape
    m = W.shape[1]
    return pl.pallas_call(
        _elu_mm_kernel,
        grid=(n // _ROW_BLK,),
        in_specs=[
            pl.BlockSpec((_ROW_BLK, k), lambda i: (i, 0)),
            pl.BlockSpec((k, m), lambda i: (0, 0)),
            pl.BlockSpec((8, m), lambda i: (0, 0)),
        ],
        out_specs=pl.BlockSpec((_ROW_BLK, m), lambda i: (i, 0)),
        out_shape=jax.ShapeDtypeStruct((n, m), jnp.float32),
    )(x, W, jnp.broadcast_to(b[None, :], (8, m)))


def _edge_sc_body(src_hbm, dst_hbm, als_hbm, ald_hbm, ex_hbm, sp_hbm,
                  srcv, dstv, alsv, aldv, exv, binsv, redv, outv, staged):
    c = lax.axis_index("c")
    s = lax.axis_index("s")
    wid = s * 2 + c
    base = wid * _PER_TILE
    pltpu.sync_copy(src_hbm.at[pl.ds(base, _PER_TILE)], srcv)
    pltpu.sync_copy(dst_hbm.at[pl.ds(base, _PER_TILE)], dstv)
    pltpu.sync_copy(als_hbm, alsv)
    pltpu.sync_copy(ald_hbm, aldv)
    zeros16 = jnp.zeros((16,), jnp.float32)

    @pl.loop(0, _NBIN, step=16)
    def _(i):
        binsv[pl.ds(i, 16)] = zeros16

    iota = lax.iota(jnp.int32, 16)
    shift_key = (iota + 15) % 16

    @pl.loop(0, _NVREG)
    def _(j):
        off = j * 16
        sidx = srcv[pl.ds(off, 16)]
        didx = dstv[pl.ds(off, 16)]
        x = plsc.load_gather(alsv, [sidx]) + plsc.load_gather(aldv, [didx])
        ex = jnp.exp(jnp.maximum(x, 0.2 * x))
        gid = (base + off) + iota
        ex = jnp.where(gid < _E, ex, 0.0)
        exv[pl.ds(off, 16)] = ex
        ks, vs = plsc.sort_key_val(didx, ex)
        cs = plsc.cumsum(vs)
        _, ks_n = plsc.sort_key_val(shift_key, ks)
        last = (ks != ks_n) | (iota == 15)
        plsc.addupdate_scatter(binsv, [ks], cs, mask=last)
        plsc.addupdate_scatter(binsv, [ks_n], -cs, mask=last & (iota < 15))

    pltpu.sync_copy(exv, ex_hbm.at[pl.ds(base, _PER_TILE)])
    pltpu.sync_copy(binsv, staged.at[s])
    plsc.subcore_barrier()
    pltpu.sync_copy(staged.at[:, pl.ds(s * _BIN_SLICE, _BIN_SLICE)], redv)

    @pl.loop(0, _BIN_SLICE, step=16)
    def _(k):
        acc = redv[0, pl.ds(k, 16)]
        for r in range(1, 16):
            acc = acc + redv[r, pl.ds(k, 16)]
        outv[pl.ds(k, 16)] = acc

    pltpu.sync_copy(outv, sp_hbm.at[c, pl.ds(s * _BIN_SLICE, _BIN_SLICE)])


def _edge_sc(src_pad, dst_pad, al_s, al_d):
    f32 = jnp.float32
    k = pl.kernel(
        _edge_sc_body,
        out_type=[
            jax.ShapeDtypeStruct((_EPAD,), f32),
            jax.ShapeDtypeStruct((2, _NBIN), f32),
        ],
        mesh=_MESH,
        compiler_params=_SC_PARAMS,
        scratch_types=[
            pltpu.VMEM((_PER_TILE,), jnp.int32),
            pltpu.VMEM((_PER_TILE,), jnp.int32),
            pltpu.VMEM((_N,), f32),
            pltpu.VMEM((_N,), f32),
            pltpu.VMEM((_PER_TILE,), f32),
            pltpu.VMEM((_NBIN,), f32),
            pltpu.VMEM((16, _BIN_SLICE), f32),
            pltpu.VMEM((_BIN_SLICE,), f32),
            pltpu.VMEM_SHARED((16, _NBIN), f32),
        ],
    )
    return k(src_pad, dst_pad, al_s, al_d)


_NBLK = _NVREG
_NROWP = 10240
_ROWS_PER_TILE = _NROWP // 16
_CHUNK = 64


def _make_agg_body(width):
  def _agg_sc_body(hc_hbm, srcm_hbm, dstm_hbm, exm_hbm, zr_hbm, part_hbm,
                 srcv, dstv, exv, gb, sb, acc, sem0, sem1, sem2, sem3):
    c = lax.axis_index("c")
    s = lax.axis_index("s")
    wid = s * 2 + c
    pltpu.sync_copy(srcm_hbm.at[wid], srcv)
    pltpu.sync_copy(dstm_hbm.at[wid], dstv)
    pltpu.sync_copy(exm_hbm.at[wid], exv)
    row0 = s * _ROWS_PER_TILE
    pltpu.sync_copy(zr_hbm.at[pl.ds(row0, _ROWS_PER_TILE)],
                    acc.at[pl.ds(row0, _ROWS_PER_TILE)])
    plsc.subcore_barrier()

    sems = [sem0, sem1, sem2, sem3]
    iota = lax.iota(jnp.int32, 16)

    def gather(b, k):
        pltpu.async_copy(hc_hbm.at[srcv.at[b]], gb.at[k], sems[k])

    def process(b, k):
        pltpu.make_async_copy(hc_hbm.at[srcv.at[b]], gb.at[k], sems[k]).wait()
        bvec = iota * 0 + b
        for r in range(16):
            ev = plsc.load_gather(exv, [bvec, jnp.full((16,), r, jnp.int32)])
            for q in range(width // 16):
                sl = pl.ds(q * 16, 16)
                sb[r, sl] = gb[k, r, sl] * ev
        pltpu.sync_copy(sb, acc.at[dstv.at[b]], add=True)

    for k in range(4):
        gather(k, k)

    @pl.loop(0, _NBLK - 1, step=4)
    def _(j):
        for k in range(4):
            b = j + k
            process(b, k)

            @pl.when(b + 4 <= _NBLK - 1)
            def _():
                gather(b + 4, k)

    process(_NBLK - 1, (_NBLK - 1) % 4)
    plsc.subcore_barrier()
    pltpu.sync_copy(acc.at[pl.ds(row0, _ROWS_PER_TILE)],
                    part_hbm.at[c, pl.ds(row0, _ROWS_PER_TILE)])
  return _agg_sc_body


def _agg_sc(h_chunk, srcm, dstm, exm, zeros_chunk):
    f32 = jnp.float32
    width = h_chunk.shape[1]
    k = pl.kernel(
        _make_agg_body(width),
        out_type=jax.ShapeDtypeStruct((2, _NROWP, width), f32),
        mesh=_MESH,
        compiler_params=_SC_PARAMS,
        scratch_types=[
            pltpu.VMEM((_NVREG, 16), jnp.int32),
            pltpu.VMEM((_NVREG, 16), jnp.int32),
            pltpu.VMEM((_NVREG, 16), f32),
            pltpu.VMEM((4, 16, width), f32),
            pltpu.VMEM((16, width), f32),
            pltpu.VMEM_SHARED((_NROWP, width), f32),
            pltpu.SemaphoreType.DMA,
            pltpu.SemaphoreType.DMA,
            pltpu.SemaphoreType.DMA,
            pltpu.SemaphoreType.DMA,
        ],
    )
    return k(h_chunk, srcm, dstm, exm, zeros_chunk)


def _mm_kernel(x_ref, w_ref, o_ref):
    o_ref[...] = jnp.dot(x_ref[...], w_ref[...],
                         precision=jax.lax.Precision.HIGHEST,
                         preferred_element_type=jnp.float32)


def _mm(x, W):
    n, k = x.shape
    m = W.shape[1]
    blk = min(n, _ROW_BLK)
    return pl.pallas_call(
        _mm_kernel,
        grid=(n // blk,),
        in_specs=[
            pl.BlockSpec((blk, k), lambda i: (i, 0)),
            pl.BlockSpec((k, m), lambda i: (0, 0)),
        ],
        out_specs=pl.BlockSpec((blk, m), lambda i: (i, 0)),
        out_shape=jax.ShapeDtypeStruct((n, m), jnp.float32),
    )(x, W)


def _hidden_kernel(p_ref, r_ref, b_ref, o_ref):
    y = (p_ref[0] + p_ref[1]) * r_ref[...] + b_ref[0:1, :]
    o_ref[...] = jnp.where(y > 0, y, jnp.exp(jnp.minimum(y, 0.0)) - 1.0)


def _hidden_combine(part, recip_mat, enc_b):
    m = enc_b.shape[0]
    return pl.pallas_call(
        _hidden_kernel,
        grid=(_N // _ROW_BLK,),
        in_specs=[
            pl.BlockSpec((2, _ROW_BLK, m), lambda i: (0, i, 0)),
            pl.BlockSpec((_ROW_BLK, m), lambda i: (i, 0)),
            pl.BlockSpec((8, m), lambda i: (0, 0)),
        ],
        out_specs=pl.BlockSpec((_ROW_BLK, m), lambda i: (i, 0)),
        out_shape=jax.ShapeDtypeStruct((_N, m), jnp.float32),
    )(part, recip_mat, jnp.broadcast_to(enc_b[None, :], (8, m)))


def _combine_out_kernel(q0_ref, q1_ref, q2_ref, q3_ref, r_ref, o_ref):
    qs = [(q[0] + q[1]) * r_ref[...]
          for q in (q0_ref, q1_ref, q2_ref, q3_ref)]
    o_ref[...] = jnp.concatenate(qs, axis=1)


def _combine_out(parts, recip_mat):
    return pl.pallas_call(
        _combine_out_kernel,
        grid=(_N // _ROW_BLK,),
        in_specs=[pl.BlockSpec((2, _ROW_BLK, _CHUNK), lambda i: (0, i, 0))] * 4
        + [pl.BlockSpec((_ROW_BLK, _CHUNK), lambda i: (i, 0))],
        out_specs=pl.BlockSpec((_ROW_BLK, 256), lambda i: (i, 0)),
        out_shape=jax.ShapeDtypeStruct((_N, 256), jnp.float32),
    )(*parts, recip_mat)


def kernel(features, edge_index, W_in, a_src_in, a_dst_in, enc_W, enc_b,
           dec_W, dec_b, W_out, a_src_out, a_dst_out):
    src = edge_index[0]
    dst = edge_index[1]
    src_pad = jnp.pad(src, (0, _EPAD - _E))
    dst_pad = jnp.pad(dst, (0, _EPAD - _E))
    srcm = src_pad.reshape(32, _NVREG, 16)
    dstm = dst_pad.reshape(32, _NVREG, 16)
    zeros_chunk = jnp.zeros((_NROWP, _CHUNK), jnp.float32)

    a_mat1 = jnp.zeros((512, 128), jnp.float32)
    a_mat1 = a_mat1.at[:, 0].set(a_src_in).at[:, 1].set(a_dst_in)
    rhs1 = jnp.concatenate([enc_W, a_mat1], axis=1)
    W1ext = _mm(W_in, rhs1)
    gal = _mm(features, W1ext)
    g1 = gal[:, :32]
    al_s = gal[:, 32]
    al_d = gal[:, 33]
    ex1, sp1 = _edge_sc(src_pad, dst_pad, al_s, al_d)
    exm1 = ex1.reshape(32, _NVREG, 16)
    recip1 = 1.0 / (sp1[0, :_N] + sp1[1, :_N] + 1e-16)
    part1 = _agg_sc(g1, srcm, dstm, exm1, jnp.zeros((_NROWP, 32), jnp.float32))
    recip1_mat = jnp.broadcast_to(recip1[:, None], (_N, 32))
    hidden_state = _hidden_combine(part1, recip1_mat, enc_b)

    d = _elu_mm(hidden_state, dec_W, dec_b)
    h2, bl_s, bl_d = _gat_project(d, W_out, a_src_out, a_dst_out)
    ex2, sp2 = _edge_sc(src_pad, dst_pad, bl_s, bl_d)
    exm2 = ex2.reshape(32, _NVREG, 16)
    recip2 = 1.0 / (sp2[0, :_N] + sp2[1, :_N] + 1e-16)
    recip2_mat = jnp.broadcast_to(recip2[:, None], (_N, _CHUNK))
    parts2 = [
        _agg_sc(h2[:, i * _CHUNK:(i + 1) * _CHUNK], srcm, dstm, exm2,
                zeros_chunk) for i in range(256 // _CHUNK)
    ]
    recon = _combine_out(parts2, recip2_mat)
    return (hidden_state, recon)
